# Initial kernel scaffold; baseline (speedup 1.0000x reference)
import jax
import jax.numpy as jnp
from jax.experimental import pallas as pl


def kernel(x, edge_index, batch, W, b):
    raise NotImplementedError("write your pallas kernel here")


# speedup vs baseline: 3.2329x; 3.2329x over previous
import jax
import jax.numpy as jnp
from jax import lax
from jax.experimental import pallas as pl
from jax.experimental.pallas import tpu as pltpu
from jax.experimental.pallas import tpu_sc as plsc

N_NODES = 10000
N_FEAT = 128
N_EDGES = 320000
N_GRAPHS = 128
N_CLASSES = 10

NT = 16
L = 16
FW = 128
CH = 128
K = 157
E_PAD = NT * K * CH
N_PAD = 10240
NODE_CHUNKS = 5
PR = 136
G_PER_TILE = N_GRAPHS // NT


def _mm_body(x_ref, w_ref, o_ref):
    o_ref[...] = jnp.dot(x_ref[...], w_ref[...],
                         preferred_element_type=jnp.float32)


def _project(xp, wp):
    return pl.pallas_call(
        _mm_body,
        out_shape=jax.ShapeDtypeStruct((N_PAD, FW), jnp.float32),
    )(xp, wp)


def _sc_body(z_hbm, row_hbm, col_hbm, bat_hbm, b_hbm, zeros_hbm, ones_hbm,
             out_hbm, h1_hbm,
             acc_s, pool_s, cnt_s,
             col_v, row_v, bat_v, gbuf, zbuf, pbuf, cbuf, obuf, bbuf):
    sid = lax.axis_index("s")
    base = sid * (N_PAD // NT)

    pltpu.sync_copy(zeros_hbm, zbuf)
    pltpu.sync_copy(b_hbm, bbuf)

    @pl.loop(0, NODE_CHUNKS)
    def _zero_acc(i):
        pltpu.sync_copy(zbuf, acc_s.at[pl.ds(base + i * CH, CH)])

    @pl.when(sid == 0)
    def _zero_pool():
        pltpu.sync_copy(zbuf, pool_s.at[pl.ds(0, CH)])
        pltpu.sync_copy(zbuf.at[pl.ds(0, PR - CH)], pool_s.at[pl.ds(CH, PR - CH)])

    @pl.when(sid == 1)
    def _zero_cnt():
        pltpu.sync_copy(zbuf, cnt_s.at[pl.ds(0, CH)])
        pltpu.sync_copy(zbuf.at[pl.ds(0, PR - CH)], cnt_s.at[pl.ds(CH, PR - CH)])

    plsc.subcore_barrier()

    def _propagate(src_hbm):
        @pl.loop(0, K)
        def _ch(j):
            eoff = (sid * K + j) * CH
            pltpu.sync_copy(col_hbm.at[pl.ds(eoff, CH)], col_v)
            pltpu.sync_copy(src_hbm.at[col_v], gbuf)
            pltpu.sync_copy(row_hbm.at[pl.ds(eoff, CH)], row_v)
            pltpu.sync_copy(gbuf, acc_s.at[row_v], add=True)

    _propagate(z_hbm)
    plsc.subcore_barrier()

    @pl.loop(0, NODE_CHUNKS)
    def _dump(i):
        span = pl.ds(base + i * CH, CH)
        pltpu.sync_copy(acc_s.at[span], gbuf)
        pltpu.sync_copy(gbuf, h1_hbm.at[span])
        pltpu.sync_copy(zbuf, acc_s.at[span])

    plsc.subcore_barrier()

    _propagate(h1_hbm)
    plsc.subcore_barrier()

    @pl.loop(0, NODE_CHUNKS)
    def _pool(i):
        noff = base + i * CH
        pltpu.sync_copy(bat_hbm.at[pl.ds(noff, CH)], bat_v)
        pltpu.sync_copy(acc_s.at[pl.ds(noff, CH)], gbuf)
        pltpu.sync_copy(gbuf, pool_s.at[bat_v], add=True)
        pltpu.sync_copy(ones_hbm, gbuf)
        pltpu.sync_copy(gbuf, cnt_s.at[bat_v], add=True)

    plsc.subcore_barrier()

    gbase = sid * G_PER_TILE
    pltpu.sync_copy(pool_s.at[pl.ds(gbase, G_PER_TILE)], pbuf)
    pltpu.sync_copy(cnt_s.at[pl.ds(gbase, G_PER_TILE)], cbuf)
    bvec = bbuf[...]
    for r in range(G_PER_TILE):
        obuf[r, :] = (pbuf[r, pl.ds(0, L)]
                      / jnp.maximum(cbuf[r, pl.ds(0, L)], 1.0) + bvec)
    pltpu.sync_copy(obuf, out_hbm.at[pl.ds(gbase, G_PER_TILE)])


def _sc_call(z, rowp, colp, batp, bp, zconst, oconst):
    mesh = plsc.VectorSubcoreMesh(
        core_axis_name="c", subcore_axis_name="s",
        num_cores=1, num_subcores=NT)
    f = pl.kernel(
        _sc_body,
        out_type=(
            jax.ShapeDtypeStruct((N_GRAPHS, L), jnp.float32),
            jax.ShapeDtypeStruct((N_PAD, FW), jnp.float32),
        ),
        mesh=mesh,
        scratch_types=[
            pltpu.VMEM_SHARED((N_PAD, FW), jnp.float32),
            pltpu.VMEM_SHARED((PR, FW), jnp.float32),
            pltpu.VMEM_SHARED((PR, FW), jnp.float32),
            pltpu.VMEM((CH,), jnp.int32),
            pltpu.VMEM((CH,), jnp.int32),
            pltpu.VMEM((CH,), jnp.int32),
            pltpu.VMEM((CH, FW), jnp.float32),
            pltpu.VMEM((CH, FW), jnp.float32),
            pltpu.VMEM((G_PER_TILE, FW), jnp.float32),
            pltpu.VMEM((G_PER_TILE, FW), jnp.float32),
            pltpu.VMEM((G_PER_TILE, L), jnp.float32),
            pltpu.VMEM((L,), jnp.float32),
        ],
    )
    out, _ = f(z, rowp, colp, batp, bp, zconst, oconst)
    return out


def kernel(x, edge_index, batch, W, b):
    f32, i32 = jnp.float32, jnp.int32
    x = x.astype(f32)
    xp = jnp.concatenate(
        [x, jnp.zeros((N_PAD - N_NODES, N_FEAT), f32)], axis=0)
    wp = jnp.zeros((N_FEAT, FW), f32).at[:, :N_CLASSES].set(
        W.astype(f32).T)
    row = edge_index[0].astype(i32)
    col = edge_index[1].astype(i32)
    pad = N_NODES + (jnp.arange(E_PAD - N_EDGES, dtype=i32) % (N_PAD - N_NODES))
    rowp = jnp.concatenate([row, pad])
    colp = jnp.concatenate([col, pad])
    batp = jnp.concatenate(
        [batch.astype(i32), jnp.full((N_PAD - N_NODES,), N_GRAPHS, i32)])
    bp = jnp.concatenate([b.astype(f32), jnp.zeros((L - N_CLASSES,), f32)])
    zconst = jnp.zeros((CH, FW), f32)
    oconst = jnp.ones((CH, FW), f32)

    z = _project(xp, wp)
    sc_out = _sc_call(z, rowp, colp, batp, bp, zconst, oconst)
    return sc_out[:, :N_CLASSES]


# speedup vs baseline: 5.3671x; 1.6601x over previous
import jax
import jax.numpy as jnp
from jax import lax
from jax.experimental import pallas as pl
from jax.experimental.pallas import tpu as pltpu
from jax.experimental.pallas import tpu_sc as plsc

N_NODES = 10000
N_FEAT = 128
N_EDGES = 320000
N_GRAPHS = 128
N_CLASSES = 10

NT = 16
L = 16
FW = 128
CH = 128
K = 158
E_PAD = NT * K * CH
N_PAD = 10240
NODE_CHUNKS = 5
ZB = 64
PR = 136
G_PER_TILE = N_GRAPHS // NT


def _mm_body(x_ref, w_ref, o_ref):
    o_ref[...] = jnp.dot(x_ref[...], w_ref[...],
                         preferred_element_type=jnp.float32)


def _project(xp, wp):
    return pl.pallas_call(
        _mm_body,
        out_shape=jax.ShapeDtypeStruct((N_PAD, FW), jnp.float32),
    )(xp, wp)


def _sc_body(z_hbm, row_hbm, col_hbm, bat_hbm, b_hbm, zeros_hbm, ones_hbm,
             out_hbm, h1_hbm,
             acc_s, pool_s, cnt_s,
             col_v0, row_v0, col_v1, row_v1, bat_v,
             gbuf0, gbuf1, zbuf, pbuf, cbuf, obuf, bbuf,
             gsem0, gsem1, ssem0, ssem1):
    sid = lax.axis_index("s")
    base = sid * (N_PAD // NT)

    pltpu.sync_copy(zeros_hbm, zbuf)
    pltpu.sync_copy(b_hbm, bbuf)

    @pl.loop(0, 2 * NODE_CHUNKS)
    def _zero_acc(i):
        pltpu.sync_copy(zbuf, acc_s.at[pl.ds(base + i * ZB, ZB)])

    @pl.when(sid == 0)
    def _zero_pool():
        pltpu.sync_copy(zbuf, pool_s.at[pl.ds(0, ZB)])
        pltpu.sync_copy(zbuf, pool_s.at[pl.ds(ZB, ZB)])
        pltpu.sync_copy(zbuf.at[pl.ds(0, PR - CH)], pool_s.at[pl.ds(CH, PR - CH)])

    @pl.when(sid == 1)
    def _zero_cnt():
        pltpu.sync_copy(zbuf, cnt_s.at[pl.ds(0, ZB)])
        pltpu.sync_copy(zbuf, cnt_s.at[pl.ds(ZB, ZB)])
        pltpu.sync_copy(zbuf.at[pl.ds(0, PR - CH)], cnt_s.at[pl.ds(CH, PR - CH)])

    plsc.subcore_barrier()

    def _propagate(src_hbm):
        ebase = sid * K * CH
        pltpu.sync_copy(col_hbm.at[pl.ds(ebase, CH)], col_v0)
        pltpu.sync_copy(row_hbm.at[pl.ds(ebase, CH)], row_v0)
        pltpu.async_copy(src_hbm.at[col_v0], gbuf0, gsem0)

        @pl.loop(0, K, step=2)
        def _pair(j):
            @pl.when(j > 0)
            def _drain1():
                pltpu.make_async_copy(gbuf1, acc_s.at[row_v1], ssem1).wait()

            eoff1 = (sid * K + j + 1) * CH
            pltpu.sync_copy(col_hbm.at[pl.ds(eoff1, CH)], col_v1)
            pltpu.sync_copy(row_hbm.at[pl.ds(eoff1, CH)], row_v1)
            pltpu.async_copy(src_hbm.at[col_v1], gbuf1, gsem1)
            pltpu.make_async_copy(src_hbm.at[col_v0], gbuf0, gsem0).wait()
            pltpu.async_copy(gbuf0, acc_s.at[row_v0], ssem0, add=True)

            pltpu.make_async_copy(gbuf0, acc_s.at[row_v0], ssem0).wait()

            @pl.when(j + 2 < K)
            def _prefetch0():
                eoff2 = (sid * K + j + 2) * CH
                pltpu.sync_copy(col_hbm.at[pl.ds(eoff2, CH)], col_v0)
                pltpu.sync_copy(row_hbm.at[pl.ds(eoff2, CH)], row_v0)
                pltpu.async_copy(src_hbm.at[col_v0], gbuf0, gsem0)

            pltpu.make_async_copy(src_hbm.at[col_v1], gbuf1, gsem1).wait()
            pltpu.async_copy(gbuf1, acc_s.at[row_v1], ssem1, add=True)

        pltpu.make_async_copy(gbuf1, acc_s.at[row_v1], ssem1).wait()

    _propagate(z_hbm)
    plsc.subcore_barrier()

    @pl.loop(0, NODE_CHUNKS)
    def _dump(i):
        span = pl.ds(base + i * CH, CH)
        pltpu.sync_copy(acc_s.at[span], gbuf0)
        pltpu.sync_copy(gbuf0, h1_hbm.at[span])
        pltpu.sync_copy(zbuf, acc_s.at[pl.ds(base + i * CH, ZB)])
        pltpu.sync_copy(zbuf, acc_s.at[pl.ds(base + i * CH + ZB, ZB)])

    plsc.subcore_barrier()

    _propagate(h1_hbm)
    plsc.subcore_barrier()

    pltpu.sync_copy(ones_hbm, gbuf1)

    @pl.loop(0, NODE_CHUNKS)
    def _pool(i):
        noff = base + i * CH
        pltpu.sync_copy(bat_hbm.at[pl.ds(noff, CH)], bat_v)
        pltpu.sync_copy(acc_s.at[pl.ds(noff, CH)], gbuf0)
        pltpu.sync_copy(gbuf0, pool_s.at[bat_v], add=True)
        pltpu.sync_copy(gbuf1, cnt_s.at[bat_v], add=True)

    plsc.subcore_barrier()

    gbase = sid * G_PER_TILE
    pltpu.sync_copy(pool_s.at[pl.ds(gbase, G_PER_TILE)], pbuf)
    pltpu.sync_copy(cnt_s.at[pl.ds(gbase, G_PER_TILE)], cbuf)
    bvec = bbuf[...]
    for r in range(G_PER_TILE):
        obuf[r, :] = (pbuf[r, pl.ds(0, L)]
                      / jnp.maximum(cbuf[r, pl.ds(0, L)], 1.0) + bvec)
    pltpu.sync_copy(obuf, out_hbm.at[pl.ds(gbase, G_PER_TILE)])


def _sc_call(z, rowp, colp, batp, bp, zconst, oconst):
    mesh = plsc.VectorSubcoreMesh(
        core_axis_name="c", subcore_axis_name="s",
        num_cores=1, num_subcores=NT)
    f = pl.kernel(
        _sc_body,
        out_type=(
            jax.ShapeDtypeStruct((N_GRAPHS, L), jnp.float32),
            jax.ShapeDtypeStruct((N_PAD, FW), jnp.float32),
        ),
        mesh=mesh,
        scratch_types=[
            pltpu.VMEM_SHARED((N_PAD, FW), jnp.float32),
            pltpu.VMEM_SHARED((PR, FW), jnp.float32),
            pltpu.VMEM_SHARED((PR, FW), jnp.float32),
            pltpu.VMEM((CH,), jnp.int32),
            pltpu.VMEM((CH,), jnp.int32),
            pltpu.VMEM((CH,), jnp.int32),
            pltpu.VMEM((CH,), jnp.int32),
            pltpu.VMEM((CH,), jnp.int32),
            pltpu.VMEM((CH, FW), jnp.float32),
            pltpu.VMEM((CH, FW), jnp.float32),
            pltpu.VMEM((ZB, FW), jnp.float32),
            pltpu.VMEM((G_PER_TILE, FW), jnp.float32),
            pltpu.VMEM((G_PER_TILE, FW), jnp.float32),
            pltpu.VMEM((G_PER_TILE, L), jnp.float32),
            pltpu.VMEM((L,), jnp.float32),
            pltpu.SemaphoreType.DMA,
            pltpu.SemaphoreType.DMA,
            pltpu.SemaphoreType.DMA,
            pltpu.SemaphoreType.DMA,
        ],
    )
    out, _ = f(z, rowp, colp, batp, bp, zconst, oconst)
    return out


def kernel(x, edge_index, batch, W, b):
    f32, i32 = jnp.float32, jnp.int32
    x = x.astype(f32)
    xp = jnp.concatenate(
        [x, jnp.zeros((N_PAD - N_NODES, N_FEAT), f32)], axis=0)
    wp = jnp.zeros((N_FEAT, FW), f32).at[:, :N_CLASSES].set(
        W.astype(f32).T)
    row = edge_index[0].astype(i32)
    col = edge_index[1].astype(i32)
    pad = N_NODES + (jnp.arange(E_PAD - N_EDGES, dtype=i32) % (N_PAD - N_NODES))
    rowp = jnp.concatenate([row, pad])
    colp = jnp.concatenate([col, pad])
    batp = jnp.concatenate(
        [batch.astype(i32), jnp.full((N_PAD - N_NODES,), N_GRAPHS, i32)])
    bp = jnp.concatenate([b.astype(f32), jnp.zeros((L - N_CLASSES,), f32)])
    zconst = jnp.zeros((ZB, FW), f32)
    oconst = jnp.ones((CH, FW), f32)

    z = _project(xp, wp)
    sc_out = _sc_call(z, rowp, colp, batp, bp, zconst, oconst)
    return sc_out[:, :N_CLASSES]


# speedup vs baseline: 8.8439x; 1.6478x over previous
import jax
import jax.numpy as jnp
from jax import lax
from jax.experimental import pallas as pl
from jax.experimental.pallas import tpu as pltpu
from jax.experimental.pallas import tpu_sc as plsc

N_NODES = 10000
N_FEAT = 128
N_EDGES = 320000
N_GRAPHS = 128
N_CLASSES = 10

NC = 2
NT = 16
L = 16
FW = 128
CH = 128
K = 80
E_PAD = NC * NT * K * CH
N_PAD = 10240
NODE_CHUNKS = 5
ZB = 64
PR = 136
G_PER_TILE = N_GRAPHS // NT


def _mm_body(x_ref, w_ref, o_ref):
    o_ref[...] = jnp.dot(x_ref[...], w_ref[...],
                         preferred_element_type=jnp.float32)


def _project(xp, wp):
    return pl.pallas_call(
        _mm_body,
        out_shape=jax.ShapeDtypeStruct((N_PAD, FW), jnp.float32),
    )(xp, wp)


def _add_body(x_ref, o_ref):
    o_ref[...] = (x_ref[pl.ds(0, N_PAD), :] + x_ref[pl.ds(N_PAD, N_PAD), :])


def _combine(hp):
    return pl.pallas_call(
        _add_body,
        out_shape=jax.ShapeDtypeStruct((N_PAD, FW), jnp.float32),
    )(hp)


def _fin_body(p_ref, c_ref, b_ref, o_ref):
    pool = (p_ref[pl.ds(0, N_GRAPHS), pl.ds(0, L)]
            + p_ref[pl.ds(N_GRAPHS, N_GRAPHS), pl.ds(0, L)])
    cnt = jnp.maximum(c_ref[:, pl.ds(0, L)], 1.0)
    o_ref[...] = pool / cnt + b_ref[...]


def _finalize(poolp, cntp, bp2):
    return pl.pallas_call(
        _fin_body,
        out_shape=jax.ShapeDtypeStruct((N_GRAPHS, L), jnp.float32),
    )(poolp, cntp, bp2)


def _zero_phase(zbuf, zeros_hbm, acc_s, base):
    pltpu.sync_copy(zeros_hbm, zbuf)

    @pl.loop(0, 2 * NODE_CHUNKS)
    def _zero_acc(i):
        pltpu.sync_copy(zbuf, acc_s.at[pl.ds(base + i * ZB, ZB)])


def _propagate(src_hbm, row_hbm, col_hbm, acc_s, wid,
               col_v0, row_v0, col_v1, row_v1, gbuf0, gbuf1,
               gsem0, gsem1, ssem0, ssem1):
    ebase = wid * K * CH
    pltpu.sync_copy(col_hbm.at[pl.ds(ebase, CH)], col_v0)
    pltpu.sync_copy(row_hbm.at[pl.ds(ebase, CH)], row_v0)
    pltpu.async_copy(src_hbm.at[col_v0], gbuf0, gsem0)

    @pl.loop(0, K, step=2)
    def _pair(j):
        @pl.when(j > 0)
        def _drain1():
            pltpu.make_async_copy(gbuf1, acc_s.at[row_v1], ssem1).wait()

        eoff1 = (wid * K + j + 1) * CH
        pltpu.sync_copy(col_hbm.at[pl.ds(eoff1, CH)], col_v1)
        pltpu.sync_copy(row_hbm.at[pl.ds(eoff1, CH)], row_v1)
        pltpu.async_copy(src_hbm.at[col_v1], gbuf1, gsem1)
        pltpu.make_async_copy(src_hbm.at[col_v0], gbuf0, gsem0).wait()
        pltpu.async_copy(gbuf0, acc_s.at[row_v0], ssem0, add=True)

        pltpu.make_async_copy(gbuf0, acc_s.at[row_v0], ssem0).wait()

        @pl.when(j + 2 < K)
        def _prefetch0():
            eoff2 = (wid * K + j + 2) * CH
            pltpu.sync_copy(col_hbm.at[pl.ds(eoff2, CH)], col_v0)
            pltpu.sync_copy(row_hbm.at[pl.ds(eoff2, CH)], row_v0)
            pltpu.async_copy(src_hbm.at[col_v0], gbuf0, gsem0)

        pltpu.make_async_copy(src_hbm.at[col_v1], gbuf1, gsem1).wait()
        pltpu.async_copy(gbuf1, acc_s.at[row_v1], ssem1, add=True)

    pltpu.make_async_copy(gbuf1, acc_s.at[row_v1], ssem1).wait()


def _sc_layer_body(src_hbm, row_hbm, col_hbm, zeros_hbm, hp_hbm,
                   acc_s,
                   col_v0, row_v0, col_v1, row_v1,
                   gbuf0, gbuf1, zbuf,
                   gsem0, gsem1, ssem0, ssem1):
    cid = lax.axis_index("c")
    sid = lax.axis_index("s")
    wid = sid * NC + cid
    base = sid * (N_PAD // NT)

    _zero_phase(zbuf, zeros_hbm, acc_s, base)
    plsc.subcore_barrier()
    _propagate(src_hbm, row_hbm, col_hbm, acc_s, wid,
               col_v0, row_v0, col_v1, row_v1, gbuf0, gbuf1,
               gsem0, gsem1, ssem0, ssem1)
    plsc.subcore_barrier()

    @pl.loop(0, NODE_CHUNKS)
    def _dump(i):
        span = pl.ds(base + i * CH, CH)
        pltpu.sync_copy(acc_s.at[span], gbuf0)
        pltpu.sync_copy(gbuf0, hp_hbm.at[pl.ds(cid * N_PAD + base + i * CH, CH)])


def _sc_layer(src, rowp, colp, zconst):
    mesh = plsc.VectorSubcoreMesh(
        core_axis_name="c", subcore_axis_name="s",
        num_cores=NC, num_subcores=NT)
    f = pl.kernel(
        _sc_layer_body,
        out_type=jax.ShapeDtypeStruct((NC * N_PAD, FW), jnp.float32),
        mesh=mesh,
        scratch_types=[
            pltpu.VMEM_SHARED((N_PAD, FW), jnp.float32),
            pltpu.VMEM((CH,), jnp.int32),
            pltpu.VMEM((CH,), jnp.int32),
            pltpu.VMEM((CH,), jnp.int32),
            pltpu.VMEM((CH,), jnp.int32),
            pltpu.VMEM((CH, FW), jnp.float32),
            pltpu.VMEM((CH, FW), jnp.float32),
            pltpu.VMEM((ZB, FW), jnp.float32),
            pltpu.SemaphoreType.DMA,
            pltpu.SemaphoreType.DMA,
            pltpu.SemaphoreType.DMA,
            pltpu.SemaphoreType.DMA,
        ],
    )
    return f(src, rowp, colp, zconst)


def _sc_layer2_body(src_hbm, row_hbm, col_hbm, bat_hbm, zeros_hbm, ones_hbm,
                    poolp_hbm, cntp_hbm,
                    acc_s, pool_s, cnt_s,
                    col_v0, row_v0, col_v1, row_v1, bat_v,
                    gbuf0, gbuf1, zbuf, pbuf,
                    gsem0, gsem1, ssem0, ssem1):
    cid = lax.axis_index("c")
    sid = lax.axis_index("s")
    wid = sid * NC + cid
    base = sid * (N_PAD // NT)

    _zero_phase(zbuf, zeros_hbm, acc_s, base)

    @pl.when(sid == 0)
    def _zero_pool():
        pltpu.sync_copy(zbuf, pool_s.at[pl.ds(0, ZB)])
        pltpu.sync_copy(zbuf, pool_s.at[pl.ds(ZB, ZB)])
        pltpu.sync_copy(zbuf.at[pl.ds(0, PR - CH)], pool_s.at[pl.ds(CH, PR - CH)])

    @pl.when(sid == 1)
    def _zero_cnt():
        pltpu.sync_copy(zbuf, cnt_s.at[pl.ds(0, ZB)])
        pltpu.sync_copy(zbuf, cnt_s.at[pl.ds(ZB, ZB)])
        pltpu.sync_copy(zbuf.at[pl.ds(0, PR - CH)], cnt_s.at[pl.ds(CH, PR - CH)])

    plsc.subcore_barrier()
    _propagate(src_hbm, row_hbm, col_hbm, acc_s, wid,
               col_v0, row_v0, col_v1, row_v1, gbuf0, gbuf1,
               gsem0, gsem1, ssem0, ssem1)
    plsc.subcore_barrier()

    pltpu.sync_copy(ones_hbm, gbuf1)

    @pl.loop(0, NODE_CHUNKS)
    def _pool(i):
        noff = base + i * CH
        pltpu.sync_copy(bat_hbm.at[pl.ds(noff, CH)], bat_v)
        pltpu.sync_copy(acc_s.at[pl.ds(noff, CH)], gbuf0)
        pltpu.sync_copy(gbuf0, pool_s.at[bat_v], add=True)
        pltpu.sync_copy(gbuf1, cnt_s.at[bat_v], add=True)

    plsc.subcore_barrier()

    gbase = sid * G_PER_TILE
    pltpu.sync_copy(pool_s.at[pl.ds(gbase, G_PER_TILE)], pbuf)
    pltpu.sync_copy(pbuf, poolp_hbm.at[pl.ds(cid * N_GRAPHS + gbase, G_PER_TILE)])

    @pl.when(cid == 0)
    def _dump_cnt():
        pltpu.sync_copy(cnt_s.at[pl.ds(gbase, G_PER_TILE)], pbuf)
        pltpu.sync_copy(pbuf, cntp_hbm.at[pl.ds(gbase, G_PER_TILE)])


def _sc_layer2(src, rowp, colp, batp, zconst, oconst):
    mesh = plsc.VectorSubcoreMesh(
        core_axis_name="c", subcore_axis_name="s",
        num_cores=NC, num_subcores=NT)
    f = pl.kernel(
        _sc_layer2_body,
        out_type=(
            jax.ShapeDtypeStruct((NC * N_GRAPHS, FW), jnp.float32),
            jax.ShapeDtypeStruct((N_GRAPHS, FW), jnp.float32),
        ),
        mesh=mesh,
        scratch_types=[
            pltpu.VMEM_SHARED((N_PAD, FW), jnp.float32),
            pltpu.VMEM_SHARED((PR, FW), jnp.float32),
            pltpu.VMEM_SHARED((PR, FW), jnp.float32),
            pltpu.VMEM((CH,), jnp.int32),
            pltpu.VMEM((CH,), jnp.int32),
            pltpu.VMEM((CH,), jnp.int32),
            pltpu.VMEM((CH,), jnp.int32),
            pltpu.VMEM((CH,), jnp.int32),
            pltpu.VMEM((CH, FW), jnp.float32),
            pltpu.VMEM((CH, FW), jnp.float32),
            pltpu.VMEM((ZB, FW), jnp.float32),
            pltpu.VMEM((G_PER_TILE, FW), jnp.float32),
            pltpu.SemaphoreType.DMA,
            pltpu.SemaphoreType.DMA,
            pltpu.SemaphoreType.DMA,
            pltpu.SemaphoreType.DMA,
        ],
    )
    return f(src, rowp, colp, batp, zconst, oconst)


def kernel(x, edge_index, batch, W, b):
    f32, i32 = jnp.float32, jnp.int32
    x = x.astype(f32)
    xp = jnp.concatenate(
        [x, jnp.zeros((N_PAD - N_NODES, N_FEAT), f32)], axis=0)
    wp = jnp.zeros((N_FEAT, FW), f32).at[:, :N_CLASSES].set(
        W.astype(f32).T)
    row = edge_index[0].astype(i32)
    col = edge_index[1].astype(i32)
    pad = N_NODES + (jnp.arange(E_PAD - N_EDGES, dtype=i32) % (N_PAD - N_NODES))
    rowp = jnp.concatenate([row, pad])
    colp = jnp.concatenate([col, pad])
    batp = jnp.concatenate(
        [batch.astype(i32), jnp.full((N_PAD - N_NODES,), N_GRAPHS, i32)])
    bp2 = jnp.concatenate(
        [b.astype(f32), jnp.zeros((L - N_CLASSES,), f32)]).reshape(1, L)
    zconst = jnp.zeros((ZB, FW), f32)
    oconst = jnp.ones((CH, FW), f32)

    z = _project(xp, wp)
    h1p = _sc_layer(z, rowp, colp, zconst)
    h1 = _combine(h1p)
    poolp, cntp = _sc_layer2(h1, rowp, colp, batp, zconst, oconst)
    out = _finalize(poolp, cntp, bp2)
    return out[:, :N_CLASSES]


# speedup vs baseline: 9.8590x; 1.1148x over previous
import jax
import jax.numpy as jnp
from jax import lax
from jax.experimental import pallas as pl
from jax.experimental.pallas import tpu as pltpu
from jax.experimental.pallas import tpu_sc as plsc

N_NODES = 10000
N_FEAT = 128
N_EDGES = 320000
N_GRAPHS = 128
N_CLASSES = 10

NC = 2
NT = 16
L = 16
FW = 128
CH = 128
K = 80
E_PAD = NC * NT * K * CH
N_PAD = 10240
NODE_CHUNKS = 5
ZB = 64
PR = 136
G_PER_TILE = N_GRAPHS // NT


def _mm_body(x_ref, w_ref, o_ref):
    o_ref[...] = jnp.dot(x_ref[...], w_ref[...],
                         preferred_element_type=jnp.float32)


def _project(xp, wp):
    return pl.pallas_call(
        _mm_body,
        out_shape=jax.ShapeDtypeStruct((N_PAD, FW), jnp.float32),
    )(xp, wp)


def _add_body(x_ref, o_ref):
    o_ref[...] = (x_ref[pl.ds(0, N_PAD), :] + x_ref[pl.ds(N_PAD, N_PAD), :])


def _combine(hp):
    return pl.pallas_call(
        _add_body,
        out_shape=jax.ShapeDtypeStruct((N_PAD, FW), jnp.float32),
    )(hp)


def _fin_body(p_ref, c_ref, b_ref, o_ref):
    pool = (p_ref[pl.ds(0, N_GRAPHS), pl.ds(0, L)]
            + p_ref[pl.ds(N_GRAPHS, N_GRAPHS), pl.ds(0, L)])
    cnt = jnp.maximum(c_ref[:, pl.ds(0, L)], 1.0)
    o_ref[...] = pool / cnt + b_ref[...]


def _finalize(poolp, cntp, bp2):
    return pl.pallas_call(
        _fin_body,
        out_shape=jax.ShapeDtypeStruct((N_GRAPHS, L), jnp.float32),
    )(poolp, cntp, bp2)


def _zero_phase(zbuf, zeros_hbm, acc_s, base):
    pltpu.sync_copy(zeros_hbm, zbuf)

    @pl.loop(0, 2 * NODE_CHUNKS)
    def _zero_acc(i):
        pltpu.sync_copy(zbuf, acc_s.at[pl.ds(base + i * ZB, ZB)])


def _propagate(src_hbm, row_hbm, col_hbm, acc_s, wid,
               col_v0, row_v0, col_v1, row_v1, gbuf0, gbuf1,
               gsem0, gsem1, ssem0, ssem1, isem0, isem1):
    ebase = wid * K * CH

    def _idx_start(eoff, col_v, row_v, isem):
        pltpu.async_copy(col_hbm.at[pl.ds(eoff, CH)], col_v, isem)
        pltpu.async_copy(row_hbm.at[pl.ds(eoff, CH)], row_v, isem)

    def _idx_wait(eoff, col_v, row_v, isem):
        pltpu.make_async_copy(col_hbm.at[pl.ds(eoff, CH)], col_v, isem).wait()
        pltpu.make_async_copy(row_hbm.at[pl.ds(eoff, CH)], row_v, isem).wait()

    _idx_start(ebase, col_v0, row_v0, isem0)
    _idx_wait(ebase, col_v0, row_v0, isem0)
    pltpu.async_copy(src_hbm.at[col_v0], gbuf0, gsem0)

    @pl.loop(0, K, step=2)
    def _pair(j):
        @pl.when(j > 0)
        def _drain1():
            pltpu.make_async_copy(gbuf1, acc_s.at[row_v1], ssem1).wait()

        eoff1 = (wid * K + j + 1) * CH
        _idx_start(eoff1, col_v1, row_v1, isem1)
        pltpu.make_async_copy(src_hbm.at[col_v0], gbuf0, gsem0).wait()
        pltpu.async_copy(gbuf0, acc_s.at[row_v0], ssem0, add=True)
        _idx_wait(eoff1, col_v1, row_v1, isem1)
        pltpu.async_copy(src_hbm.at[col_v1], gbuf1, gsem1)

        pltpu.make_async_copy(gbuf0, acc_s.at[row_v0], ssem0).wait()
        eoff2 = (wid * K + j + 2) * CH

        @pl.when(j + 2 < K)
        def _idx2_start():
            _idx_start(eoff2, col_v0, row_v0, isem0)

        pltpu.make_async_copy(src_hbm.at[col_v1], gbuf1, gsem1).wait()
        pltpu.async_copy(gbuf1, acc_s.at[row_v1], ssem1, add=True)

        @pl.when(j + 2 < K)
        def _idx2_wait():
            _idx_wait(eoff2, col_v0, row_v0, isem0)
            pltpu.async_copy(src_hbm.at[col_v0], gbuf0, gsem0)

    pltpu.make_async_copy(gbuf1, acc_s.at[row_v1], ssem1).wait()


def _sc_layer_body(src_hbm, row_hbm, col_hbm, zeros_hbm, hp_hbm,
                   acc_s,
                   col_v0, row_v0, col_v1, row_v1,
                   gbuf0, gbuf1, zbuf,
                   gsem0, gsem1, ssem0, ssem1, isem0, isem1):
    cid = lax.axis_index("c")
    sid = lax.axis_index("s")
    wid = sid * NC + cid
    base = sid * (N_PAD // NT)

    _zero_phase(zbuf, zeros_hbm, acc_s, base)
    plsc.subcore_barrier()
    _propagate(src_hbm, row_hbm, col_hbm, acc_s, wid,
               col_v0, row_v0, col_v1, row_v1, gbuf0, gbuf1,
               gsem0, gsem1, ssem0, ssem1, isem0, isem1)
    plsc.subcore_barrier()

    @pl.loop(0, NODE_CHUNKS)
    def _dump(i):
        span = pl.ds(base + i * CH, CH)
        pltpu.sync_copy(acc_s.at[span], gbuf0)
        pltpu.sync_copy(gbuf0, hp_hbm.at[pl.ds(cid * N_PAD + base + i * CH, CH)])


def _sc_layer(src, rowp, colp, zconst):
    mesh = plsc.VectorSubcoreMesh(
        core_axis_name="c", subcore_axis_name="s",
        num_cores=NC, num_subcores=NT)
    f = pl.kernel(
        _sc_layer_body,
        out_type=jax.ShapeDtypeStruct((NC * N_PAD, FW), jnp.float32),
        mesh=mesh,
        scratch_types=[
            pltpu.VMEM_SHARED((N_PAD, FW), jnp.float32),
            pltpu.VMEM((CH,), jnp.int32),
            pltpu.VMEM((CH,), jnp.int32),
            pltpu.VMEM((CH,), jnp.int32),
            pltpu.VMEM((CH,), jnp.int32),
            pltpu.VMEM((CH, FW), jnp.float32),
            pltpu.VMEM((CH, FW), jnp.float32),
            pltpu.VMEM((ZB, FW), jnp.float32),
            pltpu.SemaphoreType.DMA,
            pltpu.SemaphoreType.DMA,
            pltpu.SemaphoreType.DMA,
            pltpu.SemaphoreType.DMA,
            pltpu.SemaphoreType.DMA,
            pltpu.SemaphoreType.DMA,
        ],
    )
    return f(src, rowp, colp, zconst)


def _sc_layer2_body(src_hbm, row_hbm, col_hbm, bat_hbm, zeros_hbm, ones_hbm,
                    poolp_hbm, cntp_hbm,
                    acc_s, pool_s, cnt_s,
                    col_v0, row_v0, col_v1, row_v1, bat_v,
                    gbuf0, gbuf1, zbuf, pbuf,
                    gsem0, gsem1, ssem0, ssem1, isem0, isem1):
    cid = lax.axis_index("c")
    sid = lax.axis_index("s")
    wid = sid * NC + cid
    base = sid * (N_PAD // NT)

    _zero_phase(zbuf, zeros_hbm, acc_s, base)

    @pl.when(sid == 0)
    def _zero_pool():
        pltpu.sync_copy(zbuf, pool_s.at[pl.ds(0, ZB)])
        pltpu.sync_copy(zbuf, pool_s.at[pl.ds(ZB, ZB)])
        pltpu.sync_copy(zbuf.at[pl.ds(0, PR - CH)], pool_s.at[pl.ds(CH, PR - CH)])

    @pl.when(sid == 1)
    def _zero_cnt():
        pltpu.sync_copy(zbuf, cnt_s.at[pl.ds(0, ZB)])
        pltpu.sync_copy(zbuf, cnt_s.at[pl.ds(ZB, ZB)])
        pltpu.sync_copy(zbuf.at[pl.ds(0, PR - CH)], cnt_s.at[pl.ds(CH, PR - CH)])

    plsc.subcore_barrier()
    _propagate(src_hbm, row_hbm, col_hbm, acc_s, wid,
               col_v0, row_v0, col_v1, row_v1, gbuf0, gbuf1,
               gsem0, gsem1, ssem0, ssem1, isem0, isem1)
    plsc.subcore_barrier()

    pltpu.sync_copy(ones_hbm, gbuf1)

    @pl.loop(0, NODE_CHUNKS)
    def _pool(i):
        noff = base + i * CH
        pltpu.sync_copy(bat_hbm.at[pl.ds(noff, CH)], bat_v)
        pltpu.sync_copy(acc_s.at[pl.ds(noff, CH)], gbuf0)
        pltpu.sync_copy(gbuf0, pool_s.at[bat_v], add=True)
        pltpu.sync_copy(gbuf1, cnt_s.at[bat_v], add=True)

    plsc.subcore_barrier()

    gbase = sid * G_PER_TILE
    pltpu.sync_copy(pool_s.at[pl.ds(gbase, G_PER_TILE)], pbuf)
    pltpu.sync_copy(pbuf, poolp_hbm.at[pl.ds(cid * N_GRAPHS + gbase, G_PER_TILE)])

    @pl.when(cid == 0)
    def _dump_cnt():
        pltpu.sync_copy(cnt_s.at[pl.ds(gbase, G_PER_TILE)], pbuf)
        pltpu.sync_copy(pbuf, cntp_hbm.at[pl.ds(gbase, G_PER_TILE)])


def _sc_layer2(src, rowp, colp, batp, zconst, oconst):
    mesh = plsc.VectorSubcoreMesh(
        core_axis_name="c", subcore_axis_name="s",
        num_cores=NC, num_subcores=NT)
    f = pl.kernel(
        _sc_layer2_body,
        out_type=(
            jax.ShapeDtypeStruct((NC * N_GRAPHS, FW), jnp.float32),
            jax.ShapeDtypeStruct((N_GRAPHS, FW), jnp.float32),
        ),
        mesh=mesh,
        scratch_types=[
            pltpu.VMEM_SHARED((N_PAD, FW), jnp.float32),
            pltpu.VMEM_SHARED((PR, FW), jnp.float32),
            pltpu.VMEM_SHARED((PR, FW), jnp.float32),
            pltpu.VMEM((CH,), jnp.int32),
            pltpu.VMEM((CH,), jnp.int32),
            pltpu.VMEM((CH,), jnp.int32),
            pltpu.VMEM((CH,), jnp.int32),
            pltpu.VMEM((CH,), jnp.int32),
            pltpu.VMEM((CH, FW), jnp.float32),
            pltpu.VMEM((CH, FW), jnp.float32),
            pltpu.VMEM((ZB, FW), jnp.float32),
            pltpu.VMEM((G_PER_TILE, FW), jnp.float32),
            pltpu.SemaphoreType.DMA,
            pltpu.SemaphoreType.DMA,
            pltpu.SemaphoreType.DMA,
            pltpu.SemaphoreType.DMA,
            pltpu.SemaphoreType.DMA,
            pltpu.SemaphoreType.DMA,
        ],
    )
    return f(src, rowp, colp, batp, zconst, oconst)


def kernel(x, edge_index, batch, W, b):
    f32, i32 = jnp.float32, jnp.int32
    x = x.astype(f32)
    xp = jnp.concatenate(
        [x, jnp.zeros((N_PAD - N_NODES, N_FEAT), f32)], axis=0)
    wp = jnp.zeros((N_FEAT, FW), f32).at[:, :N_CLASSES].set(
        W.astype(f32).T)
    row = edge_index[0].astype(i32)
    col = edge_index[1].astype(i32)
    pad = N_NODES + (jnp.arange(E_PAD - N_EDGES, dtype=i32) % (N_PAD - N_NODES))
    rowp = jnp.concatenate([row, pad])
    colp = jnp.concatenate([col, pad])
    batp = jnp.concatenate(
        [batch.astype(i32), jnp.full((N_PAD - N_NODES,), N_GRAPHS, i32)])
    bp2 = jnp.concatenate(
        [b.astype(f32), jnp.zeros((L - N_CLASSES,), f32)]).reshape(1, L)
    zconst = jnp.zeros((ZB, FW), f32)
    oconst = jnp.ones((CH, FW), f32)

    z = _project(xp, wp)
    h1p = _sc_layer(z, rowp, colp, zconst)
    h1 = _combine(h1p)
    poolp, cntp = _sc_layer2(h1, rowp, colp, batp, zconst, oconst)
    out = _finalize(poolp, cntp, bp2)
    return out[:, :N_CLASSES]
